# baseline (speedup 1.0000x reference)
import functools

import jax
import jax.numpy as jnp
from jax import lax
from jax.experimental import pallas as pl
from jax.experimental.pallas import tpu as pltpu
from jax.experimental.pallas import tpu_sc as plsc

_BB = 512
_KT = 2048


def _lin(x, w):
    return lax.dot_general(x, w, (((1,), (1,)), ((), ())))


def _pre_ln_body(x_ref, wp_ref, bp_ref, wv_ref, bv_ref, wo_ref, bo_ref,
                 h2_ref):
    x = x_ref[...]
    h = _lin(x, wp_ref[...]) + bp_ref[...]
    v = _lin(h, wv_ref[...]) + bv_ref[...]
    attn_out = _lin(v, wo_ref[...]) + bo_ref[...]
    h2_ref[...] = attn_out + h


def _post_ln_body(hn_ref, w1_ref, b1_ref, w2_ref, b2_ref, wc_ref, bc_ref,
                  enc_ref):
    hn = hn_ref[...]
    m = jnp.maximum(_lin(hn, w1_ref[...]) + b1_ref[...], 0.0)
    m = jnp.maximum(_lin(m, w2_ref[...]) + b2_ref[...], 0.0)
    hm = m + hn
    enc_ref[...] = _lin(hm, wc_ref[...]) + bc_ref[...]


def _vq_body(enc_ref, encn_ref, cb_ref, cbn_ref, idx_ref, d2_ref):
    enc = enc_ref[...]
    encn = encn_ref[...]
    bb = enc.shape[0]
    k_total = cb_ref.shape[0]
    enc_s = enc * (-2.0)
    iota_f = lax.broadcasted_iota(jnp.int32, (bb, _KT), 1).astype(jnp.float32)
    big = jnp.float32(2 * k_total)

    m2 = jnp.full((bb, 1), jnp.inf, dtype=jnp.float32)
    for t in range(0, k_total, _KT):
        dot2 = _lin(enc_s, cb_ref[t:t + _KT, :])
        d2 = (encn + cbn_ref[:, t:t + _KT]) + dot2
        d2_ref[:, t:t + _KT] = d2
        m2 = jnp.minimum(m2, jnp.min(d2, axis=1, keepdims=True))

    thr = jnp.where(m2 > 0.0, m2 * (1.0 + 2.0 ** -14), 0.0)
    j1 = jnp.full((bb, 1), big, dtype=jnp.float32)
    cnt = jnp.zeros((bb, 1), dtype=jnp.float32)
    for t in range(0, k_total, _KT):
        d2 = d2_ref[:, t:t + _KT]
        mask = d2 <= thr
        cand = jnp.where(mask, iota_f, big)
        j1 = jnp.minimum(j1, jnp.min(cand, axis=1, keepdims=True)
                         + jnp.float32(t))
        cnt = cnt + jnp.sum(mask.astype(jnp.float32), axis=1, keepdims=True)
    idx_ref[0, 0, :] = j1[:, 0].astype(jnp.int32)

    @pl.when(jnp.max(cnt) > 1.5)
    def _():
        run_v = jnp.full((bb, 1), jnp.inf, dtype=jnp.float32)
        run_i = jnp.zeros((bb, 1), dtype=jnp.int32)
        for t in range(0, k_total, _KT):
            dist = jnp.sqrt(jnp.clip(d2_ref[:, t:t + _KT], 0.0))
            mv = jnp.min(dist, axis=1, keepdims=True)
            cand = jnp.where(dist == mv, iota_f, big)
            mi = jnp.min(cand, axis=1, keepdims=True).astype(jnp.int32) + t
            upd = mv < run_v
            run_i = jnp.where(upd, mi, run_i)
            run_v = jnp.where(upd, mv, run_v)
        idx_ref[0, 0, :] = run_i[:, 0]


def _dec_body(quant4_ref, idx_ref, enc_ref, wd1_ref, bd1_ref, wd2_ref, bd2_ref,
              dec_ref, loss_ref):
    i = pl.program_id(0)
    nblk = pl.num_programs(0)
    g = quant4_ref[...]
    e = enc_ref[...]
    ecols = e.shape[1]
    r = idx_ref[...] & 3
    q = jnp.where(r == 1, g[:, ecols:2 * ecols],
                  jnp.where(r == 2, g[:, 2 * ecols:3 * ecols],
                            jnp.where(r == 3, g[:, 3 * ecols:], g[:, :ecols])))
    quant_st = e + (q - e)
    d1 = jnp.maximum(_lin(quant_st, wd1_ref[...]) + bd1_ref[...], 0.0)
    d2 = jnp.maximum(_lin(d1, wd2_ref[...]) + bd2_ref[...], 0.0)
    dec_ref[...] = d2

    @pl.when(i == 0)
    def _():
        loss_ref[...] = jnp.zeros_like(loss_ref)

    loss_ref[...] = loss_ref[...] + jnp.sum((q - e) ** 2)

    @pl.when(i == nblk - 1)
    def _():
        loss_ref[...] = loss_ref[...] * (1.25 / (nblk * q.shape[0] * q.shape[1]))


def _sc_gather(codebook, idx):
    k_rows, e_dim = codebook.shape
    b = idx.shape[0]
    info = plsc.get_sparse_core_info()
    nw = info.num_cores * info.num_subcores
    b_per_w = b // nw
    mesh = plsc.VectorSubcoreMesh(core_axis_name="c", subcore_axis_name="s")

    @functools.partial(
        pl.kernel, mesh=mesh,
        out_type=jax.ShapeDtypeStruct((b, e_dim), jnp.float32),
        scratch_types=[
            pltpu.VMEM((b_per_w,), jnp.int32),
            pltpu.VMEM((b_per_w, e_dim), jnp.float32),
            pltpu.SemaphoreType.DMA,
        ],
    )
    def gather_k(table_hbm, idx_hbm, out_hbm, idx_v, rows_v, sem):
        wid = lax.axis_index("s") * info.num_cores + lax.axis_index("c")
        base = wid * b_per_w
        pltpu.sync_copy(idx_hbm.at[pl.ds(base, b_per_w)], idx_v)
        pltpu.async_copy(table_hbm.at[idx_v], rows_v, sem).wait()
        pltpu.sync_copy(rows_v, out_hbm.at[pl.ds(base, b_per_w)])

    return gather_k(codebook, idx)


def kernel(x, Wp, bp, W_in, b_in, Wo_a, bo_a, ln_g, ln_b, W1, b1, W2, b2,
           Wc, bc, codebook, Wd1, bd1, Wd2, bd2):
    B, D = x.shape
    H = Wp.shape[0]
    E = Wc.shape[0]
    K = codebook.shape[0]
    nblk = B // _BB
    row = lambda a: a.reshape(1, -1)
    Wv = W_in[2 * H:3 * H, :]
    bv = b_in[2 * H:3 * H]

    wspec = lambda shp: pl.BlockSpec(shp, lambda i: (0, 0))

    h2 = pl.pallas_call(
        _pre_ln_body,
        grid=(nblk,),
        in_specs=[pl.BlockSpec((_BB, D), lambda i: (i, 0)),
                  wspec((H, D)), wspec((1, H)), wspec((H, H)), wspec((1, H)),
                  wspec((H, H)), wspec((1, H))],
        out_specs=pl.BlockSpec((_BB, H), lambda i: (i, 0)),
        out_shape=jax.ShapeDtypeStruct((B, H), jnp.float32),
    )(x, Wp, row(bp), Wv, row(bv), Wo_a, row(bo_a))

    h2_3 = h2[:, None, :]
    mu = h2_3.mean(axis=-1, keepdims=True)
    var = h2_3.var(axis=-1, keepdims=True)
    hn = ((h2_3 - mu) / jnp.sqrt(var + 1e-5) * ln_g + ln_b)[:, 0, :]

    enc = pl.pallas_call(
        _post_ln_body,
        grid=(nblk,),
        in_specs=[pl.BlockSpec((_BB, H), lambda i: (i, 0)),
                  wspec((H, H)), wspec((1, H)), wspec((H, H)), wspec((1, H)),
                  wspec((E, H)), wspec((1, E))],
        out_specs=pl.BlockSpec((_BB, E), lambda i: (i, 0)),
        out_shape=jax.ShapeDtypeStruct((B, E), jnp.float32),
    )(hn, W1, row(b1), W2, row(b2), Wc, row(bc))

    encn = (enc ** 2).sum(-1)
    cbn = (codebook ** 2).sum(-1)

    idx3 = pl.pallas_call(
        _vq_body,
        grid=(nblk,),
        in_specs=[pl.BlockSpec((_BB, E), lambda i: (i, 0)),
                  pl.BlockSpec((_BB, 1), lambda i: (i, 0)),
                  pl.BlockSpec((K, E), lambda i: (0, 0)),
                  pl.BlockSpec((1, K), lambda i: (0, 0))],
        out_specs=pl.BlockSpec((1, 1, _BB), lambda i: (i, 0, 0)),
        out_shape=jax.ShapeDtypeStruct((nblk, 1, _BB), jnp.int32),
        scratch_shapes=[pltpu.VMEM((_BB, K), jnp.float32)],
    )(enc, encn.reshape(B, 1), codebook, row(cbn))
    idx = idx3.reshape(B)

    cb4 = codebook.reshape(K // 4, 4 * E)
    quant4 = _sc_gather(cb4, idx >> 2)

    dec, loss_arr = pl.pallas_call(
        _dec_body,
        grid=(nblk,),
        in_specs=[pl.BlockSpec((_BB, 4 * E), lambda i: (i, 0)),
                  pl.BlockSpec((_BB, 1), lambda i: (i, 0)),
                  pl.BlockSpec((_BB, E), lambda i: (i, 0)),
                  wspec((H, E)), wspec((1, H)), wspec((D, H)), wspec((1, D))],
        out_specs=[pl.BlockSpec((_BB, D), lambda i: (i, 0)),
                   pl.BlockSpec((1, 1), lambda i: (0, 0))],
        out_shape=[jax.ShapeDtypeStruct((B, D), jnp.float32),
                   jax.ShapeDtypeStruct((1, 1), jnp.float32)],
    )(quant4, idx.reshape(B, 1), enc, Wd1, row(bd1), Wd2, row(bd2))
    return dec, loss_arr[0, 0]


# speedup vs baseline: 1.1536x; 1.1536x over previous
import functools

import jax
import jax.numpy as jnp
from jax import lax
from jax.experimental import pallas as pl
from jax.experimental.pallas import tpu as pltpu
from jax.experimental.pallas import tpu_sc as plsc

_BB = 512
_KT = 2048


def _lin(x, w):
    return lax.dot_general(x, w, (((1,), (1,)), ((), ())))


def _pre_ln_body(x_ref, wp_ref, bp_ref, wv_ref, bv_ref, wo_ref, bo_ref,
                 h2_ref):
    x = x_ref[...]
    h = _lin(x, wp_ref[...]) + bp_ref[...]
    v = _lin(h, wv_ref[...]) + bv_ref[...]
    attn_out = _lin(v, wo_ref[...]) + bo_ref[...]
    h2_ref[...] = attn_out + h


def _post_ln_body(hn_ref, w1_ref, b1_ref, w2_ref, b2_ref, wc_ref, bc_ref,
                  enc_ref):
    hn = hn_ref[...]
    m = jnp.maximum(_lin(hn, w1_ref[...]) + b1_ref[...], 0.0)
    m = jnp.maximum(_lin(m, w2_ref[...]) + b2_ref[...], 0.0)
    hm = m + hn
    enc_ref[...] = _lin(hm, wc_ref[...]) + bc_ref[...]


def _vq_body(enc_ref, encn_ref, cb_ref, cbn_ref, idx_ref):
    enc = enc_ref[...]
    encn = encn_ref[...]
    bb = enc.shape[0]
    k_total = cb_ref.shape[0]
    enc_s = enc * (-2.0)
    iota_f = lax.broadcasted_iota(jnp.int32, (bb, _KT), 1).astype(jnp.float32)
    big = jnp.float32(2 * k_total)
    run_v = jnp.full((bb, 1), jnp.inf, dtype=jnp.float32)
    run_i = jnp.zeros((bb, 1), dtype=jnp.int32)
    for t in range(0, k_total, _KT):
        dot2 = _lin(enc_s, cb_ref[t:t + _KT, :])
        d2 = (encn + cbn_ref[:, t:t + _KT]) + dot2
        dist = jnp.sqrt(jnp.clip(d2, 0.0))
        mv = jnp.min(dist, axis=1, keepdims=True)
        cand = jnp.where(dist == mv, iota_f, big)
        mi = jnp.min(cand, axis=1, keepdims=True).astype(jnp.int32) + t
        upd = mv < run_v
        run_i = jnp.where(upd, mi, run_i)
        run_v = jnp.where(upd, mv, run_v)
    idx_ref[0, 0, :] = run_i[:, 0]


def _dec_body(quant4_ref, idx_ref, enc_ref, wd1_ref, bd1_ref, wd2_ref, bd2_ref,
              dec_ref, loss_ref):
    g = quant4_ref[...]
    e = enc_ref[...]
    ecols = e.shape[1]
    r = idx_ref[...] & 3
    q = jnp.where(r == 1, g[:, ecols:2 * ecols],
                  jnp.where(r == 2, g[:, 2 * ecols:3 * ecols],
                            jnp.where(r == 3, g[:, 3 * ecols:], g[:, :ecols])))
    quant_st = e + (q - e)
    d1 = jnp.maximum(_lin(quant_st, wd1_ref[...]) + bd1_ref[...], 0.0)
    d2 = jnp.maximum(_lin(d1, wd2_ref[...]) + bd2_ref[...], 0.0)
    dec_ref[...] = d2
    pos = (lax.broadcasted_iota(jnp.int32, (8, 128), 0)
           + lax.broadcasted_iota(jnp.int32, (8, 128), 1))
    loss_ref[...] = jnp.where(pos == 0, jnp.sum((q - e) ** 2), 0.0)


def _sc_gather(codebook, idx):
    k_rows, e_dim = codebook.shape
    b = idx.shape[0]
    info = plsc.get_sparse_core_info()
    nw = info.num_cores * info.num_subcores
    b_per_w = b // nw
    mesh = plsc.VectorSubcoreMesh(core_axis_name="c", subcore_axis_name="s")

    @functools.partial(
        pl.kernel, mesh=mesh,
        out_type=jax.ShapeDtypeStruct((b, e_dim), jnp.float32),
        scratch_types=[
            pltpu.VMEM((b_per_w,), jnp.int32),
            pltpu.VMEM((b_per_w, e_dim), jnp.float32),
            pltpu.SemaphoreType.DMA,
        ],
    )
    def gather_k(table_hbm, idx_hbm, out_hbm, idx_v, rows_v, sem):
        wid = lax.axis_index("s") * info.num_cores + lax.axis_index("c")
        base = wid * b_per_w
        pltpu.sync_copy(idx_hbm.at[pl.ds(base, b_per_w)], idx_v)
        pltpu.async_copy(table_hbm.at[idx_v], rows_v, sem).wait()
        pltpu.sync_copy(rows_v, out_hbm.at[pl.ds(base, b_per_w)])

    return gather_k(codebook, idx)


def kernel(x, Wp, bp, W_in, b_in, Wo_a, bo_a, ln_g, ln_b, W1, b1, W2, b2,
           Wc, bc, codebook, Wd1, bd1, Wd2, bd2):
    B, D = x.shape
    H = Wp.shape[0]
    E = Wc.shape[0]
    K = codebook.shape[0]
    nblk = B // _BB
    row = lambda a: a.reshape(1, -1)
    Wv = W_in[2 * H:3 * H, :]
    bv = b_in[2 * H:3 * H]

    wspec = lambda shp: pl.BlockSpec(shp, lambda i: (0, 0))

    h2 = pl.pallas_call(
        _pre_ln_body,
        grid=(nblk,),
        in_specs=[pl.BlockSpec((_BB, D), lambda i: (i, 0)),
                  wspec((H, D)), wspec((1, H)), wspec((H, H)), wspec((1, H)),
                  wspec((H, H)), wspec((1, H))],
        out_specs=pl.BlockSpec((_BB, H), lambda i: (i, 0)),
        out_shape=jax.ShapeDtypeStruct((B, H), jnp.float32),
        compiler_params=pltpu.CompilerParams(
            dimension_semantics=("parallel",)),
    )(x, Wp, row(bp), Wv, row(bv), Wo_a, row(bo_a))

    h2_3 = h2[:, None, :]
    mu = h2_3.mean(axis=-1, keepdims=True)
    var = h2_3.var(axis=-1, keepdims=True)
    hn = ((h2_3 - mu) / jnp.sqrt(var + 1e-5) * ln_g + ln_b)[:, 0, :]

    enc = pl.pallas_call(
        _post_ln_body,
        grid=(nblk,),
        in_specs=[pl.BlockSpec((_BB, H), lambda i: (i, 0)),
                  wspec((H, H)), wspec((1, H)), wspec((H, H)), wspec((1, H)),
                  wspec((E, H)), wspec((1, E))],
        out_specs=pl.BlockSpec((_BB, E), lambda i: (i, 0)),
        out_shape=jax.ShapeDtypeStruct((B, E), jnp.float32),
        compiler_params=pltpu.CompilerParams(
            dimension_semantics=("parallel",)),
    )(hn, W1, row(b1), W2, row(b2), Wc, row(bc))

    encn = (enc ** 2).sum(-1)
    cbn = (codebook ** 2).sum(-1)

    idx3 = pl.pallas_call(
        _vq_body,
        grid=(nblk,),
        in_specs=[pl.BlockSpec((_BB, E), lambda i: (i, 0)),
                  pl.BlockSpec((_BB, 1), lambda i: (i, 0)),
                  pl.BlockSpec((K, E), lambda i: (0, 0)),
                  pl.BlockSpec((1, K), lambda i: (0, 0))],
        out_specs=pl.BlockSpec((1, 1, _BB), lambda i: (i, 0, 0)),
        out_shape=jax.ShapeDtypeStruct((nblk, 1, _BB), jnp.int32),
        compiler_params=pltpu.CompilerParams(
            dimension_semantics=("parallel",)),
    )(enc, encn.reshape(B, 1), codebook, row(cbn))
    idx = idx3.reshape(B)

    cb4 = codebook.reshape(K // 4, 4 * E)
    quant4 = _sc_gather(cb4, idx >> 2)

    dec, loss_arr = pl.pallas_call(
        _dec_body,
        grid=(nblk,),
        in_specs=[pl.BlockSpec((_BB, 4 * E), lambda i: (i, 0)),
                  pl.BlockSpec((_BB, 1), lambda i: (i, 0)),
                  pl.BlockSpec((_BB, E), lambda i: (i, 0)),
                  wspec((H, E)), wspec((1, H)), wspec((D, H)), wspec((1, D))],
        out_specs=[pl.BlockSpec((_BB, D), lambda i: (i, 0)),
                   pl.BlockSpec((8, 128), lambda i: (i, 0))],
        out_shape=[jax.ShapeDtypeStruct((B, D), jnp.float32),
                   jax.ShapeDtypeStruct((nblk * 8, 128), jnp.float32)],
        compiler_params=pltpu.CompilerParams(
            dimension_semantics=("parallel",)),
    )(quant4, idx.reshape(B, 1), enc, Wd1, row(bd1), Wd2, row(bd2))
    loss = 1.25 * (jnp.sum(loss_arr) / (B * E))
    return dec, loss


# speedup vs baseline: 1.1724x; 1.0164x over previous
import functools

import jax
import jax.numpy as jnp
from jax import lax
from jax.experimental import pallas as pl
from jax.experimental.pallas import tpu as pltpu
from jax.experimental.pallas import tpu_sc as plsc

_BB = 512
_KT = 2048


def _lin(x, w):
    return lax.dot_general(x, w, (((1,), (1,)), ((), ())))


def _pre_ln_body(x_ref, wp_ref, bp_ref, wv_ref, bv_ref, wo_ref, bo_ref,
                 h2_ref):
    x = x_ref[...]
    h = _lin(x, wp_ref[...]) + bp_ref[...]
    v = _lin(h, wv_ref[...]) + bv_ref[...]
    attn_out = _lin(v, wo_ref[...]) + bo_ref[...]
    h2_ref[...] = attn_out + h


def _post_ln_body(h2_ref, mu_ref, var_ref, lng_ref, lnb_ref,
                  w1_ref, b1_ref, w2_ref, b2_ref, wc_ref, bc_ref,
                  enc_ref):
    h2 = h2_ref[...]
    hn = ((h2 - mu_ref[...]) / jnp.sqrt(var_ref[...] + 1e-5)
          * lng_ref[...] + lnb_ref[...])
    m = jnp.maximum(_lin(hn, w1_ref[...]) + b1_ref[...], 0.0)
    m = jnp.maximum(_lin(m, w2_ref[...]) + b2_ref[...], 0.0)
    hm = m + hn
    enc_ref[...] = _lin(hm, wc_ref[...]) + bc_ref[...]


def _vq_body(enc_ref, encn_ref, cb_ref, cbn_ref, idx_ref):
    enc = enc_ref[...]
    encn = encn_ref[...]
    bb = enc.shape[0]
    k_total = cb_ref.shape[0]
    enc_s = enc * (-2.0)
    iota_f = lax.broadcasted_iota(jnp.int32, (bb, _KT), 1).astype(jnp.float32)
    big = jnp.float32(2 * k_total)
    run_v = jnp.full((bb, 1), jnp.inf, dtype=jnp.float32)
    run_i = jnp.zeros((bb, 1), dtype=jnp.int32)
    for t in range(0, k_total, _KT):
        dot2 = _lin(enc_s, cb_ref[t:t + _KT, :])
        d2 = (encn + cbn_ref[:, t:t + _KT]) + dot2
        dist = jnp.sqrt(jnp.clip(d2, 0.0))
        mv = jnp.min(dist, axis=1, keepdims=True)
        cand = jnp.where(dist == mv, iota_f, big)
        mi = jnp.min(cand, axis=1, keepdims=True).astype(jnp.int32) + t
        upd = mv < run_v
        run_i = jnp.where(upd, mi, run_i)
        run_v = jnp.where(upd, mv, run_v)
    idx_ref[0, 0, :] = run_i[:, 0]


def _dec_body(quant4_ref, idx_ref, enc_ref, wd1_ref, bd1_ref, wd2_ref, bd2_ref,
              dec_ref, loss_ref):
    g = quant4_ref[...]
    e = enc_ref[...]
    ecols = e.shape[1]
    r = idx_ref[...] & 3
    q = jnp.where(r == 1, g[:, ecols:2 * ecols],
                  jnp.where(r == 2, g[:, 2 * ecols:3 * ecols],
                            jnp.where(r == 3, g[:, 3 * ecols:], g[:, :ecols])))
    quant_st = e + (q - e)
    d1 = jnp.maximum(_lin(quant_st, wd1_ref[...]) + bd1_ref[...], 0.0)
    d2 = jnp.maximum(_lin(d1, wd2_ref[...]) + bd2_ref[...], 0.0)
    dec_ref[...] = d2

    i = pl.program_id(0)
    nblk = pl.num_programs(0)

    @pl.when(i == 0)
    def _():
        loss_ref[...] = jnp.zeros_like(loss_ref)

    loss_ref[...] = loss_ref[...] + jnp.sum((q - e) ** 2)

    @pl.when(i == nblk - 1)
    def _():
        loss_ref[...] = loss_ref[...] * (1.25 / (nblk * q.shape[0] * ecols))


def _sc_gather(codebook, idx):
    k_rows, e_dim = codebook.shape
    b = idx.shape[0]
    info = plsc.get_sparse_core_info()
    nw = info.num_cores * info.num_subcores
    b_per_w = b // nw
    mesh = plsc.VectorSubcoreMesh(core_axis_name="c", subcore_axis_name="s")

    @functools.partial(
        pl.kernel, mesh=mesh,
        out_type=jax.ShapeDtypeStruct((b, e_dim), jnp.float32),
        scratch_types=[
            pltpu.VMEM((b_per_w,), jnp.int32),
            pltpu.VMEM((b_per_w, e_dim), jnp.float32),
            pltpu.SemaphoreType.DMA,
        ],
    )
    def gather_k(table_hbm, idx_hbm, out_hbm, idx_v, rows_v, sem):
        wid = lax.axis_index("s") * info.num_cores + lax.axis_index("c")
        base = wid * b_per_w
        pltpu.sync_copy(idx_hbm.at[pl.ds(base, b_per_w)], idx_v)
        pltpu.async_copy(table_hbm.at[idx_v], rows_v, sem).wait()
        pltpu.sync_copy(rows_v, out_hbm.at[pl.ds(base, b_per_w)])

    return gather_k(codebook, idx)


def kernel(x, Wp, bp, W_in, b_in, Wo_a, bo_a, ln_g, ln_b, W1, b1, W2, b2,
           Wc, bc, codebook, Wd1, bd1, Wd2, bd2):
    B, D = x.shape
    H = Wp.shape[0]
    E = Wc.shape[0]
    K = codebook.shape[0]
    nblk = B // _BB
    row = lambda a: a.reshape(1, -1)
    Wv = W_in[2 * H:3 * H, :]
    bv = b_in[2 * H:3 * H]

    wspec = lambda shp: pl.BlockSpec(shp, lambda i: (0, 0))

    h2 = pl.pallas_call(
        _pre_ln_body,
        grid=(nblk,),
        in_specs=[pl.BlockSpec((_BB, D), lambda i: (i, 0)),
                  wspec((H, D)), wspec((1, H)), wspec((H, H)), wspec((1, H)),
                  wspec((H, H)), wspec((1, H))],
        out_specs=pl.BlockSpec((_BB, H), lambda i: (i, 0)),
        out_shape=jax.ShapeDtypeStruct((B, H), jnp.float32),
    )(x, Wp, row(bp), Wv, row(bv), Wo_a, row(bo_a))

    mu = h2.mean(axis=-1, keepdims=True)
    var = h2.var(axis=-1, keepdims=True)

    enc = pl.pallas_call(
        _post_ln_body,
        grid=(nblk,),
        in_specs=[pl.BlockSpec((_BB, H), lambda i: (i, 0)),
                  pl.BlockSpec((_BB, 1), lambda i: (i, 0)),
                  pl.BlockSpec((_BB, 1), lambda i: (i, 0)),
                  wspec((1, H)), wspec((1, H)),
                  wspec((H, H)), wspec((1, H)), wspec((H, H)), wspec((1, H)),
                  wspec((E, H)), wspec((1, E))],
        out_specs=pl.BlockSpec((_BB, E), lambda i: (i, 0)),
        out_shape=jax.ShapeDtypeStruct((B, E), jnp.float32),
    )(h2, mu, var, row(ln_g), row(ln_b), W1, row(b1), W2, row(b2),
      Wc, row(bc))

    encn = (enc ** 2).sum(-1)
    cbn = (codebook ** 2).sum(-1)

    idx3 = pl.pallas_call(
        _vq_body,
        grid=(nblk,),
        in_specs=[pl.BlockSpec((_BB, E), lambda i: (i, 0)),
                  pl.BlockSpec((_BB, 1), lambda i: (i, 0)),
                  pl.BlockSpec((K, E), lambda i: (0, 0)),
                  pl.BlockSpec((1, K), lambda i: (0, 0))],
        out_specs=pl.BlockSpec((1, 1, _BB), lambda i: (i, 0, 0)),
        out_shape=jax.ShapeDtypeStruct((nblk, 1, _BB), jnp.int32),
    )(enc, encn.reshape(B, 1), codebook, row(cbn))
    idx = idx3.reshape(B)

    cb4 = codebook.reshape(K // 4, 4 * E)
    quant4 = _sc_gather(cb4, idx >> 2)

    dec, loss_arr = pl.pallas_call(
        _dec_body,
        grid=(nblk,),
        in_specs=[pl.BlockSpec((_BB, 4 * E), lambda i: (i, 0)),
                  pl.BlockSpec((_BB, 1), lambda i: (i, 0)),
                  pl.BlockSpec((_BB, E), lambda i: (i, 0)),
                  wspec((H, E)), wspec((1, H)), wspec((D, H)), wspec((1, D))],
        out_specs=[pl.BlockSpec((_BB, D), lambda i: (i, 0)),
                   pl.BlockSpec((1, 1), lambda i: (0, 0))],
        out_shape=[jax.ShapeDtypeStruct((B, D), jnp.float32),
                   jax.ShapeDtypeStruct((1, 1), jnp.float32)],
    )(quant4, idx.reshape(B, 1), enc, Wd1, row(bd1), Wd2, row(bd2))
    return dec, loss_arr[0, 0]


# speedup vs baseline: 1.1926x; 1.0172x over previous
import functools

import jax
import jax.numpy as jnp
from jax import lax
from jax.experimental import pallas as pl
from jax.experimental.pallas import tpu as pltpu
from jax.experimental.pallas import tpu_sc as plsc

_BB = 512
_KT = 2048


def _lin(x, w):
    return lax.dot_general(x, w, (((1,), (1,)), ((), ())))


def _pre_ln_body(x_ref, wp_ref, bp_ref, wv_ref, bv_ref, wo_ref, bo_ref,
                 h2_ref):
    x = x_ref[...]
    h = _lin(x, wp_ref[...]) + bp_ref[...]
    v = _lin(h, wv_ref[...]) + bv_ref[...]
    attn_out = _lin(v, wo_ref[...]) + bo_ref[...]
    h2_ref[...] = attn_out + h


def _post_ln_body(h2_ref, mu_ref, var_ref, lng_ref, lnb_ref,
                  w1_ref, b1_ref, w2_ref, b2_ref, wc_ref, bc_ref,
                  enc_ref):
    h2 = h2_ref[...]
    hn = ((h2 - mu_ref[...]) / jnp.sqrt(var_ref[...] + 1e-5)
          * lng_ref[...] + lnb_ref[...])
    m = jnp.maximum(_lin(hn, w1_ref[...]) + b1_ref[...], 0.0)
    m = jnp.maximum(_lin(m, w2_ref[...]) + b2_ref[...], 0.0)
    hm = m + hn
    enc_ref[...] = _lin(hm, wc_ref[...]) + bc_ref[...]


def _vq_body(enc_ref, encn_ref, cb_ref, cbn_ref, idxhi_ref, r_ref):
    enc = enc_ref[...]
    encn = encn_ref[...]
    bb = enc.shape[0]
    k_total = cb_ref.shape[0]
    enc_s = enc * (-2.0)
    iota_f = lax.broadcasted_iota(jnp.int32, (bb, _KT), 1).astype(jnp.float32)
    big = jnp.float32(2 * k_total)
    run_v = jnp.full((bb, 1), jnp.inf, dtype=jnp.float32)
    run_i = jnp.zeros((bb, 1), dtype=jnp.int32)
    for t in range(0, k_total, _KT):
        dot2 = _lin(enc_s, cb_ref[t:t + _KT, :])
        d2 = (encn + cbn_ref[:, t:t + _KT]) + dot2
        dist = jnp.sqrt(jnp.clip(d2, 0.0))
        mv = jnp.min(dist, axis=1, keepdims=True)
        cand = jnp.where(dist == mv, iota_f, big)
        mi = jnp.min(cand, axis=1, keepdims=True).astype(jnp.int32) + t
        upd = mv < run_v
        run_i = jnp.where(upd, mi, run_i)
        run_v = jnp.where(upd, mv, run_v)
    idxhi_ref[...] = (run_i[:, 0]) >> 2
    r_ref[...] = run_i & 3


def _dec_body(quant4_ref, r_ref, enc_ref, wd1_ref, bd1_ref, wd2_ref, bd2_ref,
              dec_ref, loss_ref):
    g = quant4_ref[...]
    e = enc_ref[...]
    ecols = e.shape[1]
    r = r_ref[...]
    q = jnp.where(r == 1, g[:, ecols:2 * ecols],
                  jnp.where(r == 2, g[:, 2 * ecols:3 * ecols],
                            jnp.where(r == 3, g[:, 3 * ecols:], g[:, :ecols])))
    quant_st = e + (q - e)
    d1 = jnp.maximum(_lin(quant_st, wd1_ref[...]) + bd1_ref[...], 0.0)
    d2 = jnp.maximum(_lin(d1, wd2_ref[...]) + bd2_ref[...], 0.0)
    dec_ref[...] = d2

    i = pl.program_id(0)
    nblk = pl.num_programs(0)

    @pl.when(i == 0)
    def _():
        loss_ref[...] = jnp.zeros_like(loss_ref)

    loss_ref[...] = loss_ref[...] + jnp.sum((q - e) ** 2)

    @pl.when(i == nblk - 1)
    def _():
        loss_ref[...] = loss_ref[...] * (1.25 / (nblk * q.shape[0] * ecols))


def _sc_gather(codebook, idx):
    k_rows, e_dim = codebook.shape
    b = idx.shape[0]
    info = plsc.get_sparse_core_info()
    nw = info.num_cores * info.num_subcores
    b_per_w = b // nw
    mesh = plsc.VectorSubcoreMesh(core_axis_name="c", subcore_axis_name="s")

    @functools.partial(
        pl.kernel, mesh=mesh,
        out_type=jax.ShapeDtypeStruct((b, e_dim), jnp.float32),
        scratch_types=[
            pltpu.VMEM((b_per_w,), jnp.int32),
            pltpu.VMEM((b_per_w, e_dim), jnp.float32),
            pltpu.SemaphoreType.DMA,
        ],
    )
    def gather_k(table_hbm, idx_hbm, out_hbm, idx_v, rows_v, sem):
        wid = lax.axis_index("s") * info.num_cores + lax.axis_index("c")
        base = wid * b_per_w
        pltpu.sync_copy(idx_hbm.at[pl.ds(base, b_per_w)], idx_v)
        pltpu.async_copy(table_hbm.at[idx_v], rows_v, sem).wait()
        pltpu.sync_copy(rows_v, out_hbm.at[pl.ds(base, b_per_w)])

    return gather_k(codebook, idx)


def kernel(x, Wp, bp, W_in, b_in, Wo_a, bo_a, ln_g, ln_b, W1, b1, W2, b2,
           Wc, bc, codebook, Wd1, bd1, Wd2, bd2):
    B, D = x.shape
    H = Wp.shape[0]
    E = Wc.shape[0]
    K = codebook.shape[0]
    nblk = B // _BB
    row = lambda a: a.reshape(1, -1)
    Wv = W_in[2 * H:3 * H, :]
    bv = b_in[2 * H:3 * H]

    wspec = lambda shp: pl.BlockSpec(shp, lambda i: (0, 0))

    h2 = pl.pallas_call(
        _pre_ln_body,
        grid=(nblk,),
        in_specs=[pl.BlockSpec((_BB, D), lambda i: (i, 0)),
                  wspec((H, D)), wspec((1, H)), wspec((H, H)), wspec((1, H)),
                  wspec((H, H)), wspec((1, H))],
        out_specs=pl.BlockSpec((_BB, H), lambda i: (i, 0)),
        out_shape=jax.ShapeDtypeStruct((B, H), jnp.float32),
    )(x, Wp, row(bp), Wv, row(bv), Wo_a, row(bo_a))

    mu = h2.mean(axis=-1, keepdims=True)
    var = h2.var(axis=-1, keepdims=True)

    enc = pl.pallas_call(
        _post_ln_body,
        grid=(nblk,),
        in_specs=[pl.BlockSpec((_BB, H), lambda i: (i, 0)),
                  pl.BlockSpec((_BB, 1), lambda i: (i, 0)),
                  pl.BlockSpec((_BB, 1), lambda i: (i, 0)),
                  wspec((1, H)), wspec((1, H)),
                  wspec((H, H)), wspec((1, H)), wspec((H, H)), wspec((1, H)),
                  wspec((E, H)), wspec((1, E))],
        out_specs=pl.BlockSpec((_BB, E), lambda i: (i, 0)),
        out_shape=jax.ShapeDtypeStruct((B, E), jnp.float32),
    )(h2, mu, var, row(ln_g), row(ln_b), W1, row(b1), W2, row(b2),
      Wc, row(bc))

    encn = (enc ** 2).sum(-1)
    cbn = (codebook ** 2).sum(-1)

    idx2 = pl.pallas_call(
        _vq_body,
        grid=(nblk,),
        in_specs=[pl.BlockSpec((_BB, E), lambda i: (i, 0)),
                  pl.BlockSpec((_BB, 1), lambda i: (i, 0)),
                  pl.BlockSpec((K, E), lambda i: (0, 0)),
                  pl.BlockSpec((1, K), lambda i: (0, 0))],
        out_specs=[pl.BlockSpec((_BB,), lambda i: (i,)),
                   pl.BlockSpec((_BB, 1), lambda i: (i, 0))],
        out_shape=[jax.ShapeDtypeStruct((B,), jnp.int32),
                   jax.ShapeDtypeStruct((B, 1), jnp.int32)],
    )(enc, encn.reshape(B, 1), codebook, row(cbn))
    idx_hi, idx_r = idx2

    cb4 = codebook.reshape(K // 4, 4 * E)
    quant4 = _sc_gather(cb4, idx_hi)

    dec, loss_arr = pl.pallas_call(
        _dec_body,
        grid=(nblk,),
        in_specs=[pl.BlockSpec((_BB, 4 * E), lambda i: (i, 0)),
                  pl.BlockSpec((_BB, 1), lambda i: (i, 0)),
                  pl.BlockSpec((_BB, E), lambda i: (i, 0)),
                  wspec((H, E)), wspec((1, H)), wspec((D, H)), wspec((1, D))],
        out_specs=[pl.BlockSpec((_BB, D), lambda i: (i, 0)),
                   pl.BlockSpec((1, 1), lambda i: (0, 0))],
        out_shape=[jax.ShapeDtypeStruct((B, D), jnp.float32),
                   jax.ShapeDtypeStruct((1, 1), jnp.float32)],
    )(quant4, idx_r, enc, Wd1, row(bd1), Wd2, row(bd2))
    return dec, loss_arr[0, 0]


# speedup vs baseline: 1.2144x; 1.0183x over previous
import functools

import jax
import jax.numpy as jnp
from jax import lax
from jax.experimental import pallas as pl
from jax.experimental.pallas import tpu as pltpu
from jax.experimental.pallas import tpu_sc as plsc

_BB = 512
_BBQ = 1024
_KT = 2048


def _lin(x, w):
    return lax.dot_general(x, w, (((1,), (1,)), ((), ())))


def _pre_ln_body(x_ref, wp_ref, bp_ref, wv_ref, bv_ref, wo_ref, bo_ref,
                 h2_ref):
    x = x_ref[...]
    h = _lin(x, wp_ref[...]) + bp_ref[...]
    v = _lin(h, wv_ref[...]) + bv_ref[...]
    attn_out = _lin(v, wo_ref[...]) + bo_ref[...]
    h2_ref[...] = attn_out + h


def _post_ln_body(h2_ref, mu_ref, var_ref, lng_ref, lnb_ref,
                  w1_ref, b1_ref, w2_ref, b2_ref, wc_ref, bc_ref,
                  enc_ref):
    h2 = h2_ref[...]
    hn = ((h2 - mu_ref[...]) / jnp.sqrt(var_ref[...] + 1e-5)
          * lng_ref[...] + lnb_ref[...])
    m = jnp.maximum(_lin(hn, w1_ref[...]) + b1_ref[...], 0.0)
    m = jnp.maximum(_lin(m, w2_ref[...]) + b2_ref[...], 0.0)
    hm = m + hn
    enc_ref[...] = _lin(hm, wc_ref[...]) + bc_ref[...]


def _vq_body(enc_ref, encn_ref, cb_ref, cbn_ref, idxhi_ref, r_ref):
    enc = enc_ref[...]
    encn = encn_ref[...]
    bb = enc.shape[0]
    k_total = cb_ref.shape[0]
    enc_s = enc * (-2.0)
    iota_f = lax.broadcasted_iota(jnp.int32, (bb, _KT), 1).astype(jnp.float32)
    big = jnp.float32(2 * k_total)
    run_v = jnp.full((bb, 1), jnp.inf, dtype=jnp.float32)
    run_i = jnp.zeros((bb, 1), dtype=jnp.int32)
    for t in range(0, k_total, _KT):
        dot2 = _lin(enc_s, cb_ref[t:t + _KT, :])
        d2 = (encn + cbn_ref[:, t:t + _KT]) + dot2
        dist = jnp.sqrt(jnp.clip(d2, 0.0))
        mv = jnp.min(dist, axis=1, keepdims=True)
        cand = jnp.where(dist == mv, iota_f, big)
        mi = jnp.min(cand, axis=1, keepdims=True).astype(jnp.int32) + t
        upd = mv < run_v
        run_i = jnp.where(upd, mi, run_i)
        run_v = jnp.where(upd, mv, run_v)
    idxhi_ref[...] = (run_i[:, 0]) >> 2
    r_ref[...] = run_i & 3


def _dec_body(quant4_ref, r_ref, enc_ref, wd1_ref, bd1_ref, wd2_ref, bd2_ref,
              dec_ref, loss_ref):
    g = quant4_ref[...]
    e = enc_ref[...]
    ecols = e.shape[1]
    r = r_ref[...]
    q = jnp.where(r == 1, g[:, ecols:2 * ecols],
                  jnp.where(r == 2, g[:, 2 * ecols:3 * ecols],
                            jnp.where(r == 3, g[:, 3 * ecols:], g[:, :ecols])))
    quant_st = e + (q - e)
    d1 = jnp.maximum(_lin(quant_st, wd1_ref[...]) + bd1_ref[...], 0.0)
    d2 = jnp.maximum(_lin(d1, wd2_ref[...]) + bd2_ref[...], 0.0)
    dec_ref[...] = d2

    i = pl.program_id(0)
    nblk = pl.num_programs(0)

    @pl.when(i == 0)
    def _():
        loss_ref[...] = jnp.zeros_like(loss_ref)

    loss_ref[...] = loss_ref[...] + jnp.sum((q - e) ** 2)

    @pl.when(i == nblk - 1)
    def _():
        loss_ref[...] = loss_ref[...] * (1.25 / (nblk * q.shape[0] * ecols))


def _sc_gather(codebook, idx):
    k_rows, e_dim = codebook.shape
    b = idx.shape[0]
    info = plsc.get_sparse_core_info()
    nw = info.num_cores * info.num_subcores
    b_per_w = b // nw
    mesh = plsc.VectorSubcoreMesh(core_axis_name="c", subcore_axis_name="s")

    @functools.partial(
        pl.kernel, mesh=mesh,
        out_type=jax.ShapeDtypeStruct((b, e_dim), jnp.float32),
        scratch_types=[
            pltpu.VMEM((b_per_w,), jnp.int32),
            pltpu.VMEM((b_per_w, e_dim), jnp.float32),
            pltpu.SemaphoreType.DMA,
        ],
    )
    def gather_k(table_hbm, idx_hbm, out_hbm, idx_v, rows_v, sem):
        wid = lax.axis_index("s") * info.num_cores + lax.axis_index("c")
        base = wid * b_per_w
        pltpu.sync_copy(idx_hbm.at[pl.ds(base, b_per_w)], idx_v)
        pltpu.async_copy(table_hbm.at[idx_v], rows_v, sem).wait()
        pltpu.sync_copy(rows_v, out_hbm.at[pl.ds(base, b_per_w)])

    return gather_k(codebook, idx)


def kernel(x, Wp, bp, W_in, b_in, Wo_a, bo_a, ln_g, ln_b, W1, b1, W2, b2,
           Wc, bc, codebook, Wd1, bd1, Wd2, bd2):
    B, D = x.shape
    H = Wp.shape[0]
    E = Wc.shape[0]
    K = codebook.shape[0]
    nblk = B // _BB
    row = lambda a: a.reshape(1, -1)
    Wv = W_in[2 * H:3 * H, :]
    bv = b_in[2 * H:3 * H]

    wspec = lambda shp: pl.BlockSpec(shp, lambda i: (0, 0))

    h2 = pl.pallas_call(
        _pre_ln_body,
        grid=(nblk,),
        in_specs=[pl.BlockSpec((_BB, D), lambda i: (i, 0)),
                  wspec((H, D)), wspec((1, H)), wspec((H, H)), wspec((1, H)),
                  wspec((H, H)), wspec((1, H))],
        out_specs=pl.BlockSpec((_BB, H), lambda i: (i, 0)),
        out_shape=jax.ShapeDtypeStruct((B, H), jnp.float32),
    )(x, Wp, row(bp), Wv, row(bv), Wo_a, row(bo_a))

    mu = h2.mean(axis=-1, keepdims=True)
    var = h2.var(axis=-1, keepdims=True)

    enc = pl.pallas_call(
        _post_ln_body,
        grid=(nblk,),
        in_specs=[pl.BlockSpec((_BB, H), lambda i: (i, 0)),
                  pl.BlockSpec((_BB, 1), lambda i: (i, 0)),
                  pl.BlockSpec((_BB, 1), lambda i: (i, 0)),
                  wspec((1, H)), wspec((1, H)),
                  wspec((H, H)), wspec((1, H)), wspec((H, H)), wspec((1, H)),
                  wspec((E, H)), wspec((1, E))],
        out_specs=pl.BlockSpec((_BB, E), lambda i: (i, 0)),
        out_shape=jax.ShapeDtypeStruct((B, E), jnp.float32),
    )(h2, mu, var, row(ln_g), row(ln_b), W1, row(b1), W2, row(b2),
      Wc, row(bc))

    encn = (enc ** 2).sum(-1)
    cbn = (codebook ** 2).sum(-1)

    idx2 = pl.pallas_call(
        _vq_body,
        grid=(B // _BBQ,),
        in_specs=[pl.BlockSpec((_BBQ, E), lambda i: (i, 0)),
                  pl.BlockSpec((_BBQ, 1), lambda i: (i, 0)),
                  pl.BlockSpec((K, E), lambda i: (0, 0)),
                  pl.BlockSpec((1, K), lambda i: (0, 0))],
        out_specs=[pl.BlockSpec((_BBQ,), lambda i: (i,)),
                   pl.BlockSpec((_BBQ, 1), lambda i: (i, 0))],
        out_shape=[jax.ShapeDtypeStruct((B,), jnp.int32),
                   jax.ShapeDtypeStruct((B, 1), jnp.int32)],
    )(enc, encn.reshape(B, 1), codebook, row(cbn))
    idx_hi, idx_r = idx2

    cb4 = codebook.reshape(K // 4, 4 * E)
    quant4 = _sc_gather(cb4, idx_hi)

    dec, loss_arr = pl.pallas_call(
        _dec_body,
        grid=(nblk,),
        in_specs=[pl.BlockSpec((_BB, 4 * E), lambda i: (i, 0)),
                  pl.BlockSpec((_BB, 1), lambda i: (i, 0)),
                  pl.BlockSpec((_BB, E), lambda i: (i, 0)),
                  wspec((H, E)), wspec((1, H)), wspec((D, H)), wspec((1, D))],
        out_specs=[pl.BlockSpec((_BB, D), lambda i: (i, 0)),
                   pl.BlockSpec((1, 1), lambda i: (0, 0))],
        out_shape=[jax.ShapeDtypeStruct((B, D), jnp.float32),
                   jax.ShapeDtypeStruct((1, 1), jnp.float32)],
    )(quant4, idx_r, enc, Wd1, row(bd1), Wd2, row(bd2))
    return dec, loss_arr[0, 0]
